# baseline (speedup 1.0000x reference)
import jax
import jax.numpy as jnp
from jax import lax
from jax.experimental import pallas as pl
from jax.experimental.pallas import tpu as pltpu
from jax.experimental.pallas import tpu_sc as plsc

B = 4096
T = 200
D = 32
NW = 32
CB = 512
NBB = B // CB
NTASK = T * NBB
PER_W = NTASK // NW
PADW = CB + 1
NBUF = 3


def _sc_body(xt_hbm, tok_hbm, post_hbm, col_hbm, out_hbm,
             idx0, idx1, idx2, rows0, rows1, rows2, outt0, outt1, outt2,
             post_v, col_v,
             s_i0, s_i1, s_i2, s_g0, s_g1, s_g2, s_s0, s_s1, s_s2):
    wid = lax.axis_index("s") * 2 + lax.axis_index("c")
    base_task = wid * PER_W

    pltpu.sync_copy(post_hbm, post_v)
    pltpu.sync_copy(col_hbm, col_v)

    iota16 = lax.iota(jnp.int32, 16)
    iota16b = iota16 + 16

    idx = (idx0, idx1, idx2)
    rows = (rows0, rows1, rows2)
    outt = (outt0, outt1, outt2)
    s_i = (s_i0, s_i1, s_i2)
    s_g = (s_g0, s_g1, s_g2)
    s_s = (s_s0, s_s1, s_s2)

    def task_tb(i):
        tk = base_task + i
        return tk >> 3, pl.multiple_of((tk & 7) << 9, CB)

    def start_idx(i):
        t, b0 = task_tb(i)
        return pltpu.async_copy(
            xt_hbm.at[t, pl.ds(b0, CB)], idx[i % NBUF], s_i[i % NBUF])

    def start_gather(i):
        return pltpu.async_copy(
            tok_hbm.at[idx[i % NBUF]], rows[i % NBUF], s_g[i % NBUF])

    def start_scatter(i):
        t, b0 = task_tb(i)
        return pltpu.async_copy(
            outt[i % NBUF].at[:, pl.ds(0, CB)],
            out_hbm.at[t, :, pl.ds(b0, CB)], s_s[i % NBUF])

    idx_h = {}
    gat_h = {}
    sct_h = {}

    idx_h[0] = start_idx(0)
    idx_h[1] = start_idx(1)
    idx_h[0].wait()
    gat_h[0] = start_gather(0)
    idx_h[1].wait()
    gat_h[1] = start_gather(1)
    idx_h[2] = start_idx(2)

    for i in range(PER_W):
        gat_h[i].wait()
        if i + 2 < PER_W:
            idx_h[i + 2].wait()
            if i >= 1:
                sct_h[i - 1].wait()
            gat_h[i + 2] = start_gather(i + 2)
            if i + 3 < PER_W:
                idx_h[i + 3] = start_idx(i + 3)

        t, _ = task_tb(i)
        f = (t * 3277) >> 16
        tspl = jnp.full((16,), t, jnp.int32)
        fspl = jnp.full((16,), f, jnp.int32)
        a0 = (plsc.load_gather(post_v, [iota16, tspl])
              + plsc.load_gather(col_v, [fspl, iota16]))
        a1 = (plsc.load_gather(post_v, [iota16b, tspl])
              + plsc.load_gather(col_v, [fspl, iota16b]))

        p = i % NBUF

        def _tr(j4, carry, p=p, a0=a0, a1=a1):
            j = j4 * 4
            for u in range(4):
                ju = j + u
                v0 = rows[p][ju, pl.ds(0, 16)] + a0
                v1 = rows[p][ju, pl.ds(16, 16)] + a1
                jspl = jnp.full((16,), 0, jnp.int32) + ju
                plsc.store_scatter(outt[p], [iota16, jspl], v0)
                plsc.store_scatter(outt[p], [iota16b, jspl], v1)
            return carry

        lax.fori_loop(0, CB // 4, _tr, 0)
        sct_h[i] = start_scatter(i)

    sct_h[PER_W - 2].wait()
    sct_h[PER_W - 1].wait()


VG = 1000000
CCH = 256
NCONV = 999936 // CCH
NPAIR = 61


def _conv_body(tokt_hbm, tbl_hbm, cin0, cin1, ct0, ct1,
               s_c0, s_c1, s_w0, s_w1):
    wid = lax.axis_index("s") * 2 + lax.axis_index("c")
    iota16 = lax.iota(jnp.int32, 16)
    cin = (cin0, cin1)
    ct = (ct0, ct1)
    s_c = (s_c0, s_c1)
    s_w = (s_w0, s_w1)

    def conv_in(c, p):
        return pltpu.async_copy(
            tokt_hbm.at[:, pl.ds(pl.multiple_of(c * CCH, CCH), CCH)],
            cin[p], s_c[p])

    def conv_out(c, p):
        return pltpu.async_copy(
            ct[p].at[pl.ds(0, CCH), pl.ds(0, D)],
            tbl_hbm.at[pl.ds(pl.multiple_of(c * CCH, CCH), CCH), :],
            s_w[p])

    def wait_in(p):
        pltpu.make_async_copy(
            tokt_hbm.at[:, pl.ds(0, CCH)], cin[p], s_c[p]).wait()

    def wait_out(p):
        pltpu.make_async_copy(
            ct[p].at[pl.ds(0, CCH), pl.ds(0, D)],
            tbl_hbm.at[pl.ds(0, CCH), :], s_w[p]).wait()

    def transpose_chunk(p, nk=CCH // 16):
        def _row(d, carry, p=p, nk=nk):
            dspl = jnp.full((16,), 0, jnp.int32) + d
            for k in range(nk):
                v = cin[p][d, pl.ds(k * 16, 16)]
                plsc.store_scatter(ct[p], [iota16 + (k * 16), dspl], v)
            return carry
        lax.fori_loop(0, D, _row, 0)

    conv_in(wid, 0)

    def conv_pair(j, carry):
        c0 = wid + (2 * j) * NW
        c1 = wid + (2 * j + 1) * NW
        conv_in(c1, 1)
        wait_in(0)

        @pl.when(j >= 1)
        def _():
            wait_out(0)

        transpose_chunk(0)
        conv_out(c0, 0)

        @pl.when(j + 1 < NPAIR)
        def _():
            conv_in(wid + (2 * j + 2) * NW, 0)

        wait_in(1)

        @pl.when(j >= 1)
        def _():
            wait_out(1)

        transpose_chunk(1)
        conv_out(c1, 1)
        return carry

    lax.fori_loop(0, NPAIR, conv_pair, 0)
    wait_out(0)
    wait_out(1)

    @pl.when(wid < 2)
    def _():
        conv_in(wid + 2 * NPAIR * NW, 0)
        wait_in(0)
        transpose_chunk(0)
        conv_out(wid + 2 * NPAIR * NW, 0)
        wait_out(0)

    @pl.when(wid == 2)
    def _():
        pltpu.async_copy(
            tokt_hbm.at[:, pl.ds(999936, 64)],
            cin0.at[:, pl.ds(0, 64)], s_c0).wait()

        def _row(d, carry):
            dspl = jnp.full((16,), 0, jnp.int32) + d
            for k in range(4):
                v = cin0[d, pl.ds(k * 16, 16)]
                plsc.store_scatter(ct0, [iota16 + (k * 16), dspl], v)
            return carry
        lax.fori_loop(0, D, _row, 0)
        pltpu.async_copy(
            ct0.at[pl.ds(0, 64), pl.ds(0, D)],
            tbl_hbm.at[pl.ds(999936, 64), :], s_w0).wait()


@jax.jit
def _sc_convert(tokt):
    mesh = plsc.VectorSubcoreMesh(core_axis_name="c", subcore_axis_name="s")
    f = pl.kernel(
        _conv_body,
        mesh=mesh,
        out_type=jax.ShapeDtypeStruct((VG, D), jnp.float32),
        compiler_params=pltpu.CompilerParams(
            use_tc_tiling_on_sc=False, needs_layout_passes=False),
        scratch_types=[
            pltpu.VMEM((D, CCH), jnp.float32),
            pltpu.VMEM((D, CCH), jnp.float32),
            pltpu.VMEM((CCH, D + 1), jnp.float32),
            pltpu.VMEM((CCH, D + 1), jnp.float32),
            pltpu.SemaphoreType.DMA,
            pltpu.SemaphoreType.DMA,
            pltpu.SemaphoreType.DMA,
            pltpu.SemaphoreType.DMA,
        ],
    )
    return f(tokt)


@jax.jit
def _sc_kernel(xt, token_table, post, col_table):
    mesh = plsc.VectorSubcoreMesh(core_axis_name="c", subcore_axis_name="s")
    f = pl.kernel(
        _sc_body,
        mesh=mesh,
        out_type=jax.ShapeDtypeStruct((T, D, B), jnp.float32),
        compiler_params=pltpu.CompilerParams(
            use_tc_tiling_on_sc=False, needs_layout_passes=False),
        scratch_types=[
            pltpu.VMEM((CB,), jnp.int32),
            pltpu.VMEM((CB,), jnp.int32),
            pltpu.VMEM((CB,), jnp.int32),
            pltpu.VMEM((CB, D), jnp.float32),
            pltpu.VMEM((CB, D), jnp.float32),
            pltpu.VMEM((CB, D), jnp.float32),
            pltpu.VMEM((D, PADW), jnp.float32),
            pltpu.VMEM((D, PADW), jnp.float32),
            pltpu.VMEM((D, PADW), jnp.float32),
            pltpu.VMEM((D, T), jnp.float32),
            pltpu.VMEM((11, D), jnp.float32),
            pltpu.SemaphoreType.DMA,
            pltpu.SemaphoreType.DMA,
            pltpu.SemaphoreType.DMA,
            pltpu.SemaphoreType.DMA,
            pltpu.SemaphoreType.DMA,
            pltpu.SemaphoreType.DMA,
            pltpu.SemaphoreType.DMA,
            pltpu.SemaphoreType.DMA,
            pltpu.SemaphoreType.DMA,
        ],
    )
    return f(xt, token_table, post, col_table)


def kernel(x, token_table, pos_table, col_table):
    xt = x.T.astype(jnp.int32)
    post = pos_table.T
    tbl = _sc_convert(token_table.T)
    outt = _sc_kernel(xt, tbl, post, col_table)
    return outt.transpose(2, 0, 1)


# speedup vs baseline: 4.1881x; 4.1881x over previous
import jax
import jax.numpy as jnp
from jax import lax
from jax.experimental import pallas as pl
from jax.experimental.pallas import tpu as pltpu
from jax.experimental.pallas import tpu_sc as plsc

B = 4096
T = 200
D = 32
NW = 32
CB = 512
NBB = B // CB
NTASK = T * NBB
PER_W = NTASK // NW
PADW = CB + 1
NBUF = 3


def _sc_body(xt_hbm, tok_hbm, post_hbm, col_hbm, out_hbm,
             idx0, idx1, idx2, rows0, rows1, rows2, outt0, outt1, outt2,
             post_v, col_v,
             s_i0, s_i1, s_i2, s_g0, s_g1, s_g2, s_s0, s_s1, s_s2):
    wid = lax.axis_index("s") * 2 + lax.axis_index("c")
    base_task = wid * PER_W

    pltpu.sync_copy(post_hbm, post_v)
    pltpu.sync_copy(col_hbm, col_v)

    iota16 = lax.iota(jnp.int32, 16)
    iota16b = iota16 + 16

    idx = (idx0, idx1, idx2)
    rows = (rows0, rows1, rows2)
    outt = (outt0, outt1, outt2)
    s_i = (s_i0, s_i1, s_i2)
    s_g = (s_g0, s_g1, s_g2)
    s_s = (s_s0, s_s1, s_s2)

    def task_tb(i):
        tk = base_task + i
        return tk >> 3, pl.multiple_of((tk & 7) << 9, CB)

    def start_idx(i):
        t, b0 = task_tb(i)
        return pltpu.async_copy(
            xt_hbm.at[t, pl.ds(b0, CB)], idx[i % NBUF], s_i[i % NBUF])

    def start_gather(i):
        return pltpu.async_copy(
            tok_hbm.at[idx[i % NBUF]], rows[i % NBUF], s_g[i % NBUF])

    def start_scatter(i):
        t, b0 = task_tb(i)
        return pltpu.async_copy(
            outt[i % NBUF].at[:, pl.ds(0, CB)],
            out_hbm.at[t, :, pl.ds(b0, CB)], s_s[i % NBUF])

    idx_h = {}
    gat_h = {}
    sct_h = {}

    idx_h[0] = start_idx(0)
    idx_h[1] = start_idx(1)
    idx_h[0].wait()
    gat_h[0] = start_gather(0)
    idx_h[1].wait()
    gat_h[1] = start_gather(1)
    idx_h[2] = start_idx(2)

    for i in range(PER_W):
        gat_h[i].wait()
        if i + 2 < PER_W:
            idx_h[i + 2].wait()
            if i >= 1:
                sct_h[i - 1].wait()
            gat_h[i + 2] = start_gather(i + 2)
            if i + 3 < PER_W:
                idx_h[i + 3] = start_idx(i + 3)

        t, _ = task_tb(i)
        f = (t * 3277) >> 16
        tspl = jnp.full((16,), t, jnp.int32)
        fspl = jnp.full((16,), f, jnp.int32)
        a0 = (plsc.load_gather(post_v, [iota16, tspl])
              + plsc.load_gather(col_v, [fspl, iota16]))
        a1 = (plsc.load_gather(post_v, [iota16b, tspl])
              + plsc.load_gather(col_v, [fspl, iota16b]))

        p = i % NBUF

        def _tr(j4, carry, p=p, a0=a0, a1=a1):
            j = j4 * 4
            for u in range(4):
                ju = j + u
                v0 = rows[p][ju, pl.ds(0, 16)] + a0
                v1 = rows[p][ju, pl.ds(16, 16)] + a1
                jspl = jnp.full((16,), 0, jnp.int32) + ju
                plsc.store_scatter(outt[p], [iota16, jspl], v0)
                plsc.store_scatter(outt[p], [iota16b, jspl], v1)
            return carry

        lax.fori_loop(0, CB // 4, _tr, 0)
        sct_h[i] = start_scatter(i)

    sct_h[PER_W - 2].wait()
    sct_h[PER_W - 1].wait()


@jax.jit
def _sc_kernel(xt, token_table, post, col_table):
    mesh = plsc.VectorSubcoreMesh(core_axis_name="c", subcore_axis_name="s")
    f = pl.kernel(
        _sc_body,
        mesh=mesh,
        out_type=jax.ShapeDtypeStruct((T, D, B), jnp.float32),
        compiler_params=pltpu.CompilerParams(
            use_tc_tiling_on_sc=False, needs_layout_passes=False),
        scratch_types=[
            pltpu.VMEM((CB,), jnp.int32),
            pltpu.VMEM((CB,), jnp.int32),
            pltpu.VMEM((CB,), jnp.int32),
            pltpu.VMEM((CB, D), jnp.float32),
            pltpu.VMEM((CB, D), jnp.float32),
            pltpu.VMEM((CB, D), jnp.float32),
            pltpu.VMEM((D, PADW), jnp.float32),
            pltpu.VMEM((D, PADW), jnp.float32),
            pltpu.VMEM((D, PADW), jnp.float32),
            pltpu.VMEM((D, T), jnp.float32),
            pltpu.VMEM((11, D), jnp.float32),
            pltpu.SemaphoreType.DMA,
            pltpu.SemaphoreType.DMA,
            pltpu.SemaphoreType.DMA,
            pltpu.SemaphoreType.DMA,
            pltpu.SemaphoreType.DMA,
            pltpu.SemaphoreType.DMA,
            pltpu.SemaphoreType.DMA,
            pltpu.SemaphoreType.DMA,
            pltpu.SemaphoreType.DMA,
        ],
    )
    return f(xt, token_table, post, col_table)


def kernel(x, token_table, pos_table, col_table):
    xt = x.T.astype(jnp.int32)
    post = pos_table.T
    outt = _sc_kernel(xt, token_table, post, col_table)
    return outt.transpose(2, 0, 1)


# speedup vs baseline: 4.7637x; 1.1374x over previous
import jax
import jax.numpy as jnp
from jax import lax
from jax.experimental import pallas as pl
from jax.experimental.pallas import tpu as pltpu
from jax.experimental.pallas import tpu_sc as plsc

B = 4096
T = 200
D = 32
NW = 32
CB = 512
NBB = B // CB
NTASK = T * NBB
PER_W = NTASK // NW
NBUF = 3


def _sc_body(xt_hbm, tok_hbm, post_hbm, col_hbm, out_hbm,
             idx0, idx1, idx2, rows0, rows1, rows2, outt0, outt1, outt2,
             post_v, col_v,
             s_i0, s_i1, s_i2, s_g0, s_g1, s_g2, s_s0, s_s1, s_s2):
    wid = lax.axis_index("s") * 2 + lax.axis_index("c")
    base_task = wid * PER_W

    pltpu.sync_copy(post_hbm, post_v)
    pltpu.sync_copy(col_hbm, col_v)

    iota16 = lax.iota(jnp.int32, 16)
    iota16b = iota16 + 16
    db_lo = iota16 >> 3
    db_hi = db_lo + 2
    di_v = iota16 & 7

    idx = (idx0, idx1, idx2)
    rows = (rows0, rows1, rows2)
    outt = (outt0, outt1, outt2)
    s_i = (s_i0, s_i1, s_i2)
    s_g = (s_g0, s_g1, s_g2)
    s_s = (s_s0, s_s1, s_s2)

    def task_tb(i):
        tk = base_task + i
        return tk >> 3, pl.multiple_of((tk & 7) << 9, CB)

    def start_idx(i):
        t, b0 = task_tb(i)
        return pltpu.async_copy(
            xt_hbm.at[t, pl.ds(b0, CB)], idx[i % NBUF], s_i[i % NBUF])

    def start_gather(i):
        return pltpu.async_copy(
            tok_hbm.at[idx[i % NBUF]], rows[i % NBUF], s_g[i % NBUF])

    def start_scatter(i):
        t, b0 = task_tb(i)
        bb0 = pl.multiple_of(b0 >> 7, 4)
        return pltpu.async_copy(
            outt[i % NBUF].at[:, pl.ds(0, 4), :, pl.ds(0, 128)],
            out_hbm.at[t, :, pl.ds(bb0, 4), :, :], s_s[i % NBUF])

    idx_h = {}
    gat_h = {}
    sct_h = {}

    idx_h[0] = start_idx(0)
    idx_h[1] = start_idx(1)
    idx_h[0].wait()
    gat_h[0] = start_gather(0)
    idx_h[1].wait()
    gat_h[1] = start_gather(1)
    idx_h[2] = start_idx(2)

    for i in range(PER_W):
        gat_h[i].wait()
        if i + 2 < PER_W:
            idx_h[i + 2].wait()
            if i >= 1:
                sct_h[i - 1].wait()
            gat_h[i + 2] = start_gather(i + 2)
            if i + 3 < PER_W:
                idx_h[i + 3] = start_idx(i + 3)

        t, _ = task_tb(i)
        f = (t * 3277) >> 16
        tspl = jnp.full((16,), t, jnp.int32)
        fspl = jnp.full((16,), f, jnp.int32)
        a0 = (plsc.load_gather(post_v, [iota16, tspl])
              + plsc.load_gather(col_v, [fspl, iota16]))
        a1 = (plsc.load_gather(post_v, [iota16b, tspl])
              + plsc.load_gather(col_v, [fspl, iota16b]))

        p = i % NBUF

        def _tr(j4, carry, p=p, a0=a0, a1=a1):
            j = j4 * 4
            for u in range(4):
                ju = j + u
                v0 = rows[p][ju, pl.ds(0, 16)] + a0
                v1 = rows[p][ju, pl.ds(16, 16)] + a1
                bbspl = jnp.full((16,), 0, jnp.int32) + (ju >> 7)
                bispl = jnp.full((16,), 0, jnp.int32) + (ju & 127)
                plsc.store_scatter(outt[p], [db_lo, bbspl, di_v, bispl], v0)
                plsc.store_scatter(outt[p], [db_hi, bbspl, di_v, bispl], v1)
            return carry

        lax.fori_loop(0, CB // 4, _tr, 0)
        sct_h[i] = start_scatter(i)

    sct_h[PER_W - 2].wait()
    sct_h[PER_W - 1].wait()


@jax.jit
def _sc_kernel(xt, token_table, post, col_table):
    mesh = plsc.VectorSubcoreMesh(core_axis_name="c", subcore_axis_name="s")
    f = pl.kernel(
        _sc_body,
        mesh=mesh,
        out_type=jax.ShapeDtypeStruct((T, D // 8, B // 128, 8, 128), jnp.float32),
        compiler_params=pltpu.CompilerParams(
            use_tc_tiling_on_sc=False, needs_layout_passes=False),
        scratch_types=[
            pltpu.VMEM((CB,), jnp.int32),
            pltpu.VMEM((CB,), jnp.int32),
            pltpu.VMEM((CB,), jnp.int32),
            pltpu.VMEM((CB, D), jnp.float32),
            pltpu.VMEM((CB, D), jnp.float32),
            pltpu.VMEM((CB, D), jnp.float32),
            pltpu.VMEM((4, 5, 8, 129), jnp.float32),
            pltpu.VMEM((4, 5, 8, 129), jnp.float32),
            pltpu.VMEM((4, 5, 8, 129), jnp.float32),
            pltpu.VMEM((D, T), jnp.float32),
            pltpu.VMEM((11, D), jnp.float32),
            pltpu.SemaphoreType.DMA,
            pltpu.SemaphoreType.DMA,
            pltpu.SemaphoreType.DMA,
            pltpu.SemaphoreType.DMA,
            pltpu.SemaphoreType.DMA,
            pltpu.SemaphoreType.DMA,
            pltpu.SemaphoreType.DMA,
            pltpu.SemaphoreType.DMA,
            pltpu.SemaphoreType.DMA,
        ],
    )
    return f(xt, token_table, post, col_table)


def kernel(x, token_table, pos_table, col_table):
    xt = x.T.astype(jnp.int32)
    post = pos_table.T
    out5 = _sc_kernel(xt, token_table, post, col_table)
    return out5.transpose(2, 4, 0, 1, 3).reshape(B, T, D)
